# baseline (device time: 82726 ns/iter reference)
import jax
import jax.numpy as jnp
from jax import lax
from jax.experimental import pallas as pl
from jax.experimental.pallas import tpu as pltpu

N_DEV = 4
M_PER = 2048
K = 1024
N = 1024
HALF = M_PER // 2
CHUNK = HALF // N_DEV
N_HOP = N_DEV - 1
SUBS = 2
SUB = CHUNK // SUBS


def kernel(t, W):
    def body(
        t_ref, w_ref, out_ref,
        t_vmem, w_vmem,
        accT, recvT, agT,
        accB, recvB, agB,
        send_sems, recv_sems, local_sems,
    ):
        my = lax.axis_index("i")
        left = lax.rem(my + N_DEV - 1, N_DEV)
        right = lax.rem(my + 1, N_DEV)

        def rs_chunk(ring_idx, j):
            if ring_idx == 0:
                return lax.rem(my + N_DEV - j, N_DEV)
            return lax.rem(my + j, N_DEV)

        def t_rows(ring_idx, c):
            return pl.ds(ring_idx * HALF + c * CHUNK, CHUNK)

        t_copies = {}
        for j in range(N_DEV):
            for r in (0, 1):
                c = rs_chunk(r, j)
                cp = pltpu.make_async_copy(
                    t_ref.at[t_rows(r, c), :],
                    t_vmem.at[t_rows(r, c), :],
                    local_sems.at[r * N_DEV + j],
                )
                cp.start()
                t_copies[(r, j)] = cp
        w_copy = pltpu.make_async_copy(w_ref, w_vmem, local_sems.at[8])
        w_copy.start()

        barrier_sem = pltpu.get_barrier_semaphore()
        for nbr in (left, right):
            pl.semaphore_signal(
                barrier_sem, inc=1,
                device_id=(nbr,), device_id_type=pl.DeviceIdType.MESH,
            )
        pl.semaphore_wait(barrier_sem, 2)

        rings = (
            (accT, recvT, agT, 0, right),
            (accB, recvB, agB, 1, left),
        )

        def t_sub(ring_idx, c, s):
            return t_vmem[pl.ds(ring_idx * HALF + c * CHUNK + s * SUB, SUB), :]

        def out_sub_rows(ring_idx, c, s):
            return pl.ds(ring_idx * HALF + c * CHUNK + s * SUB, SUB)

        def sem_idx(phase, ring_idx, h, s):
            return ((phase * 2 + ring_idx) * N_HOP + h) * SUBS + s

        def make_rs(acc, recv, ring_idx, dst, h, s):
            return pltpu.make_async_remote_copy(
                src_ref=acc.at[s],
                dst_ref=recv.at[h, s],
                send_sem=send_sems.at[sem_idx(0, ring_idx, h, s)],
                recv_sem=recv_sems.at[sem_idx(0, ring_idx, h, s)],
                device_id=(dst,), device_id_type=pl.DeviceIdType.MESH,
            )

        def make_ag(ag, ring_idx, dst, h, s):
            return pltpu.make_async_remote_copy(
                src_ref=ag.at[h, s],
                dst_ref=ag.at[h + 1, s],
                send_sem=send_sems.at[sem_idx(1, ring_idx, h, s)],
                recv_sem=recv_sems.at[sem_idx(1, ring_idx, h, s)],
                device_id=(dst,), device_id_type=pl.DeviceIdType.MESH,
            )

        rs_d = {}
        ag_d = {}
        out_d = []

        for acc, recv, ag, r, dst in rings:
            t_copies[(r, 0)].wait()
            for s in range(SUBS):
                acc[s] = t_sub(r, my, s)
                d = rs_d[(r, 0, s)] = make_rs(acc, recv, r, dst, 0, s)
                d.start()

        for h in range(N_HOP):
            for acc, recv, ag, r, dst in rings:
                t_copies[(r, h + 1)].wait()
            if h + 1 == N_HOP:
                w_copy.wait()
            for s in range(SUBS):
                for acc, recv, ag, r, dst in rings:
                    d = rs_d[(r, h, s)]
                    d.wait_recv()
                    d.wait_send()
                    nxt = rs_chunk(r, h + 1)
                    acc[s] = recv[h, s] + t_sub(r, nxt, s)
                    if h + 1 < N_HOP:
                        d2 = rs_d[(r, h + 1, s)] = make_rs(acc, recv, r, dst, h + 1, s)
                        d2.start()
                    else:
                        y = jnp.dot(
                            acc[s], w_vmem[...],
                            preferred_element_type=jnp.float32,
                        )
                        ag[0, s] = y
                        d2 = ag_d[(r, 0, s)] = make_ag(ag, r, dst, 0, s)
                        d2.start()
                        st = pltpu.make_async_copy(
                            ag.at[0, s],
                            out_ref.at[out_sub_rows(r, nxt, s), :],
                            local_sems.at[9 + (r * (1 + N_HOP)) * SUBS + s],
                        )
                        st.start()
                        out_d.append(st)

        for h in range(N_HOP):
            for s in range(SUBS):
                for acc, recv, ag, r, dst in rings:
                    d = ag_d[(r, h, s)]
                    d.wait_recv()
                    if h + 1 < N_HOP:
                        d2 = ag_d[(r, h + 1, s)] = make_ag(ag, r, dst, h + 1, s)
                        d2.start()
                    orig = rs_chunk(r, h)
                    st = pltpu.make_async_copy(
                        ag.at[h + 1, s],
                        out_ref.at[out_sub_rows(r, orig, s), :],
                        local_sems.at[9 + (r * (1 + N_HOP) + 1 + h) * SUBS + s],
                    )
                    st.start()
                    out_d.append(st)

        for h in range(N_HOP):
            for s in range(SUBS):
                for r in (0, 1):
                    ag_d[(r, h, s)].wait_send()
        for st in out_d:
            st.wait()

    return pl.pallas_call(
        body,
        out_shape=jax.ShapeDtypeStruct((M_PER, N), jnp.float32),
        in_specs=[
            pl.BlockSpec(memory_space=pltpu.MemorySpace.HBM),
            pl.BlockSpec(memory_space=pltpu.MemorySpace.HBM),
        ],
        out_specs=pl.BlockSpec(memory_space=pltpu.MemorySpace.HBM),
        scratch_shapes=[
            pltpu.VMEM((M_PER, K), jnp.float32),
            pltpu.VMEM((K, N), jnp.float32),
            pltpu.VMEM((SUBS, SUB, K), jnp.float32),
            pltpu.VMEM((N_HOP, SUBS, SUB, K), jnp.float32),
            pltpu.VMEM((N_DEV, SUBS, SUB, N), jnp.float32),
            pltpu.VMEM((SUBS, SUB, K), jnp.float32),
            pltpu.VMEM((N_HOP, SUBS, SUB, K), jnp.float32),
            pltpu.VMEM((N_DEV, SUBS, SUB, N), jnp.float32),
            pltpu.SemaphoreType.DMA((2 * 2 * N_HOP * SUBS,)),
            pltpu.SemaphoreType.DMA((2 * 2 * N_HOP * SUBS,)),
            pltpu.SemaphoreType.DMA((9 + 2 * (1 + N_HOP) * SUBS,)),
        ],
        compiler_params=pltpu.CompilerParams(collective_id=0),
    )(t, W)


# device time: 81604 ns/iter; 1.0137x vs baseline; 1.0137x over previous
import jax
import jax.numpy as jnp
from jax import lax
from jax.experimental import pallas as pl
from jax.experimental.pallas import tpu as pltpu

N_DEV = 4
M_PER = 2048
K = 1024
N = 1024
HALF = M_PER // 2
CHUNK = HALF // N_DEV
N_HOP = N_DEV - 1
SUBS = 2
SUB = CHUNK // SUBS


def kernel(t, W):
    def body(
        t_ref, w_ref, out_ref,
        t_vmem, w_vmem,
        accT, recvT, agT,
        accB, recvB, agB,
        send_sems, recv_sems, local_sems,
    ):
        my = lax.axis_index("i")
        left = lax.rem(my + N_DEV - 1, N_DEV)
        right = lax.rem(my + 1, N_DEV)

        def rs_chunk(ring_idx, j):
            if ring_idx == 0:
                return lax.rem(my + N_DEV - j, N_DEV)
            return lax.rem(my + j, N_DEV)

        def t_rows(ring_idx, c):
            return pl.ds(ring_idx * HALF + c * CHUNK, CHUNK)

        t_copies = {}
        for j in range(N_DEV):
            for r in (0, 1):
                c = rs_chunk(r, j)
                cp = pltpu.make_async_copy(
                    t_ref.at[t_rows(r, c), :],
                    t_vmem.at[t_rows(r, c), :],
                    local_sems.at[r * N_DEV + j],
                )
                cp.start()
                t_copies[(r, j)] = cp
        w_copy = pltpu.make_async_copy(w_ref, w_vmem, local_sems.at[8])
        w_copy.start()

        barrier_sem = pltpu.get_barrier_semaphore()
        for nbr in (left, right):
            pl.semaphore_signal(
                barrier_sem, inc=1,
                device_id=(nbr,), device_id_type=pl.DeviceIdType.MESH,
            )
        pl.semaphore_wait(barrier_sem, 2)

        rings = (
            (accT, recvT, agT, 0, right),
            (accB, recvB, agB, 1, left),
        )

        def t_sub(ring_idx, c, s):
            return t_vmem[pl.ds(ring_idx * HALF + c * CHUNK + s * SUB, SUB), :]

        def out_sub_rows(ring_idx, c, s):
            return pl.ds(ring_idx * HALF + c * CHUNK + s * SUB, SUB)

        def sem_idx(phase, ring_idx, h, s):
            return ((phase * 2 + ring_idx) * N_HOP + h) * SUBS + s

        def make_rs(acc, recv, ring_idx, dst, h, s, src=None):
            return pltpu.make_async_remote_copy(
                src_ref=acc.at[s] if src is None else src,
                dst_ref=recv.at[h, s],
                send_sem=send_sems.at[sem_idx(0, ring_idx, h, s)],
                recv_sem=recv_sems.at[sem_idx(0, ring_idx, h, s)],
                device_id=(dst,), device_id_type=pl.DeviceIdType.MESH,
            )

        def make_ag(ag, ring_idx, dst, h, s):
            return pltpu.make_async_remote_copy(
                src_ref=ag.at[h, s],
                dst_ref=ag.at[h + 1, s],
                send_sem=send_sems.at[sem_idx(1, ring_idx, h, s)],
                recv_sem=recv_sems.at[sem_idx(1, ring_idx, h, s)],
                device_id=(dst,), device_id_type=pl.DeviceIdType.MESH,
            )

        rs_d = {}
        ag_d = {}
        out_d = []

        for acc, recv, ag, r, dst in rings:
            t_copies[(r, 0)].wait()
            for s in range(SUBS):
                src = t_vmem.at[
                    pl.ds(r * HALF + my * CHUNK + s * SUB, SUB), :
                ]
                d = rs_d[(r, 0, s)] = make_rs(acc, recv, r, dst, 0, s, src=src)
                d.start()

        for h in range(N_HOP):
            for acc, recv, ag, r, dst in rings:
                t_copies[(r, h + 1)].wait()
            if h + 1 == N_HOP:
                w_copy.wait()
            for s in range(SUBS):
                for acc, recv, ag, r, dst in rings:
                    d = rs_d[(r, h, s)]
                    d.wait_recv()
                    d.wait_send()
                    nxt = rs_chunk(r, h + 1)
                    acc[s] = recv[h, s] + t_sub(r, nxt, s)
                    if h + 1 < N_HOP:
                        d2 = rs_d[(r, h + 1, s)] = make_rs(acc, recv, r, dst, h + 1, s)
                        d2.start()
                    else:
                        y = jnp.dot(
                            acc[s], w_vmem[...],
                            preferred_element_type=jnp.float32,
                        )
                        ag[0, s] = y
                        d2 = ag_d[(r, 0, s)] = make_ag(ag, r, dst, 0, s)
                        d2.start()
                        st = pltpu.make_async_copy(
                            ag.at[0, s],
                            out_ref.at[out_sub_rows(r, nxt, s), :],
                            local_sems.at[9 + (r * (1 + N_HOP)) * SUBS + s],
                        )
                        st.start()
                        out_d.append(st)

        for h in range(N_HOP):
            for s in range(SUBS):
                for acc, recv, ag, r, dst in rings:
                    d = ag_d[(r, h, s)]
                    d.wait_recv()
                    if h + 1 < N_HOP:
                        d2 = ag_d[(r, h + 1, s)] = make_ag(ag, r, dst, h + 1, s)
                        d2.start()
                    orig = rs_chunk(r, h)
                    st = pltpu.make_async_copy(
                        ag.at[h + 1, s],
                        out_ref.at[out_sub_rows(r, orig, s), :],
                        local_sems.at[9 + (r * (1 + N_HOP) + 1 + h) * SUBS + s],
                    )
                    st.start()
                    out_d.append(st)

        for h in range(N_HOP):
            for s in range(SUBS):
                for r in (0, 1):
                    ag_d[(r, h, s)].wait_send()
        for st in out_d:
            st.wait()

    return pl.pallas_call(
        body,
        out_shape=jax.ShapeDtypeStruct((M_PER, N), jnp.float32),
        in_specs=[
            pl.BlockSpec(memory_space=pltpu.MemorySpace.HBM),
            pl.BlockSpec(memory_space=pltpu.MemorySpace.HBM),
        ],
        out_specs=pl.BlockSpec(memory_space=pltpu.MemorySpace.HBM),
        scratch_shapes=[
            pltpu.VMEM((M_PER, K), jnp.float32),
            pltpu.VMEM((K, N), jnp.float32),
            pltpu.VMEM((SUBS, SUB, K), jnp.float32),
            pltpu.VMEM((N_HOP, SUBS, SUB, K), jnp.float32),
            pltpu.VMEM((N_DEV, SUBS, SUB, N), jnp.float32),
            pltpu.VMEM((SUBS, SUB, K), jnp.float32),
            pltpu.VMEM((N_HOP, SUBS, SUB, K), jnp.float32),
            pltpu.VMEM((N_DEV, SUBS, SUB, N), jnp.float32),
            pltpu.SemaphoreType.DMA((2 * 2 * N_HOP * SUBS,)),
            pltpu.SemaphoreType.DMA((2 * 2 * N_HOP * SUBS,)),
            pltpu.SemaphoreType.DMA((9 + 2 * (1 + N_HOP) * SUBS,)),
        ],
        input_output_aliases={0: 0},
        compiler_params=pltpu.CompilerParams(collective_id=0),
    )(t, W)


# device time: 81564 ns/iter; 1.0142x vs baseline; 1.0005x over previous
import jax
import jax.numpy as jnp
from jax import lax
from jax.experimental import pallas as pl
from jax.experimental.pallas import tpu as pltpu

N_DEV = 4
M_PER = 2048
K = 1024
N = 1024
HALF = M_PER // 2
CHUNK = HALF // N_DEV
N_HOP = N_DEV - 1
SUBS = 2
SUB = CHUNK // SUBS


def kernel(t, W):
    def body(
        t_ref, w_ref, out_ref,
        t_vmem, w_vmem,
        accT, recvT, agT,
        accB, recvB, agB,
        send_sems, recv_sems, local_sems,
    ):
        my = lax.axis_index("i")
        left = lax.rem(my + N_DEV - 1, N_DEV)
        right = lax.rem(my + 1, N_DEV)

        def rs_chunk(ring_idx, j):
            if ring_idx == 0:
                return lax.rem(my + N_DEV - j, N_DEV)
            return lax.rem(my + j, N_DEV)

        def t_rows(ring_idx, c):
            return pl.ds(ring_idx * HALF + c * CHUNK, CHUNK)

        t_copies = {}
        for j in range(N_DEV):
            for r in (0, 1):
                c = rs_chunk(r, j)
                cp = pltpu.make_async_copy(
                    t_ref.at[t_rows(r, c), :],
                    t_vmem.at[t_rows(r, c), :],
                    local_sems.at[r * N_DEV + j],
                )
                cp.start()
                t_copies[(r, j)] = cp
        w_copy = pltpu.make_async_copy(w_ref, w_vmem, local_sems.at[8])
        w_copy.start()

        with jax.named_scope("barrier"):
            barrier_sem = pltpu.get_barrier_semaphore()
            for nbr in (left, right):
                pl.semaphore_signal(
                    barrier_sem, inc=1,
                    device_id=(nbr,), device_id_type=pl.DeviceIdType.MESH,
                )
            pl.semaphore_wait(barrier_sem, 2)

        rings = (
            (accT, recvT, agT, 0, right),
            (accB, recvB, agB, 1, left),
        )

        def t_sub(ring_idx, c, s):
            return t_vmem[pl.ds(ring_idx * HALF + c * CHUNK + s * SUB, SUB), :]

        def out_sub_rows(ring_idx, c, s):
            return pl.ds(ring_idx * HALF + c * CHUNK + s * SUB, SUB)

        def sem_idx(phase, ring_idx, h, s):
            return ((phase * 2 + ring_idx) * N_HOP + h) * SUBS + s

        def make_rs(acc, recv, ring_idx, dst, h, s, src=None):
            return pltpu.make_async_remote_copy(
                src_ref=acc.at[s] if src is None else src,
                dst_ref=recv.at[h, s],
                send_sem=send_sems.at[sem_idx(0, ring_idx, h, s)],
                recv_sem=recv_sems.at[sem_idx(0, ring_idx, h, s)],
                device_id=(dst,), device_id_type=pl.DeviceIdType.MESH,
            )

        def make_ag(ag, ring_idx, dst, h, s):
            return pltpu.make_async_remote_copy(
                src_ref=ag.at[h, s],
                dst_ref=ag.at[h + 1, s],
                send_sem=send_sems.at[sem_idx(1, ring_idx, h, s)],
                recv_sem=recv_sems.at[sem_idx(1, ring_idx, h, s)],
                device_id=(dst,), device_id_type=pl.DeviceIdType.MESH,
            )

        rs_d = {}
        ag_d = {}
        out_d = []

        for acc, recv, ag, r, dst in rings:
            t_copies[(r, 0)].wait()
            for s in range(SUBS):
                src = t_vmem.at[
                    pl.ds(r * HALF + my * CHUNK + s * SUB, SUB), :
                ]
                d = rs_d[(r, 0, s)] = make_rs(acc, recv, r, dst, 0, s, src=src)
                d.start()

        for h in range(N_HOP):
          with jax.named_scope(f"rs#hop={h}"):
            for acc, recv, ag, r, dst in rings:
                t_copies[(r, h + 1)].wait()
            if h + 1 == N_HOP:
                w_copy.wait()
            for s in range(SUBS):
                for acc, recv, ag, r, dst in rings:
                    d = rs_d[(r, h, s)]
                    d.wait_recv()
                    d.wait_send()
                    nxt = rs_chunk(r, h + 1)
                    acc[s] = recv[h, s] + t_sub(r, nxt, s)
                    if h + 1 < N_HOP:
                        d2 = rs_d[(r, h + 1, s)] = make_rs(acc, recv, r, dst, h + 1, s)
                        d2.start()
                    else:
                        y = jnp.dot(
                            acc[s], w_vmem[...],
                            preferred_element_type=jnp.float32,
                        )
                        ag[0, s] = y
                        d2 = ag_d[(r, 0, s)] = make_ag(ag, r, dst, 0, s)
                        d2.start()
                        st = pltpu.make_async_copy(
                            ag.at[0, s],
                            out_ref.at[out_sub_rows(r, nxt, s), :],
                            local_sems.at[9 + (r * (1 + N_HOP)) * SUBS + s],
                        )
                        st.start()
                        out_d.append(st)

        for h in range(N_HOP):
          with jax.named_scope(f"ag#hop={h}"):
            for s in range(SUBS):
                for acc, recv, ag, r, dst in rings:
                    d = ag_d[(r, h, s)]
                    d.wait_recv()
                    if h + 1 < N_HOP:
                        d2 = ag_d[(r, h + 1, s)] = make_ag(ag, r, dst, h + 1, s)
                        d2.start()
                    orig = rs_chunk(r, h)
                    st = pltpu.make_async_copy(
                        ag.at[h + 1, s],
                        out_ref.at[out_sub_rows(r, orig, s), :],
                        local_sems.at[9 + (r * (1 + N_HOP) + 1 + h) * SUBS + s],
                    )
                    st.start()
                    out_d.append(st)

        with jax.named_scope("drain"):
            for h in range(N_HOP):
                for s in range(SUBS):
                    for r in (0, 1):
                        ag_d[(r, h, s)].wait_send()
            for st in out_d:
                st.wait()

    return pl.pallas_call(
        body,
        out_shape=jax.ShapeDtypeStruct((M_PER, N), jnp.float32),
        in_specs=[
            pl.BlockSpec(memory_space=pltpu.MemorySpace.HBM),
            pl.BlockSpec(memory_space=pltpu.MemorySpace.HBM),
        ],
        out_specs=pl.BlockSpec(memory_space=pltpu.MemorySpace.HBM),
        scratch_shapes=[
            pltpu.VMEM((M_PER, K), jnp.float32),
            pltpu.VMEM((K, N), jnp.float32),
            pltpu.VMEM((SUBS, SUB, K), jnp.float32),
            pltpu.VMEM((N_HOP, SUBS, SUB, K), jnp.float32),
            pltpu.VMEM((N_DEV, SUBS, SUB, N), jnp.float32),
            pltpu.VMEM((SUBS, SUB, K), jnp.float32),
            pltpu.VMEM((N_HOP, SUBS, SUB, K), jnp.float32),
            pltpu.VMEM((N_DEV, SUBS, SUB, N), jnp.float32),
            pltpu.SemaphoreType.DMA((2 * 2 * N_HOP * SUBS,)),
            pltpu.SemaphoreType.DMA((2 * 2 * N_HOP * SUBS,)),
            pltpu.SemaphoreType.DMA((9 + 2 * (1 + N_HOP) * SUBS,)),
        ],
        input_output_aliases={0: 0},
        compiler_params=pltpu.CompilerParams(collective_id=0),
    )(t, W)
